# baseline (speedup 1.0000x reference)
import functools

import jax
import jax.numpy as jnp
from jax import lax
from jax.experimental import pallas as pl
from jax.experimental.pallas import tpu as pltpu
from jax.experimental.pallas import tpu_sc as plsc

B, H, W, D = 2, 128, 128, 128
BI, BJ = 8, 8
HALO = 5
WY, WX = BI + 2 * HALO, BJ + 2 * HALO
NTI, NTJ = H // BI, W // BJ
NTILES = B * NTI * NTJ
NC, NS = 2, 16
NW = NC * NS
TPW = NTILES // NW
WROW = WX * D
WSIZE = WY * WROW
FROW = BJ * D * 3
OROW = BJ * D
KV = D // 16
NVEC = BI * BJ * KV
YLO_MAX = H - WY


def _axis(p, lo_f, lo_s, hi_s, wm1):
    t = p.astype(jnp.int32)
    tf = t.astype(jnp.float32)
    d = jnp.minimum(tf + 1.0 - p, 1.0)
    u = tf - lo_f
    u1 = u + 1.0
    c0 = jnp.minimum(jnp.maximum(u, 0.0), wm1).astype(jnp.int32)
    c1 = jnp.minimum(jnp.maximum(u1, 0.0), wm1).astype(jnp.int32)
    bad = (u < lo_s) | (u1 > hi_s)
    return d, c0, c1, bad, tf


def _zaxis(p):
    t = p.astype(jnp.int32)
    tf = t.astype(jnp.float32)
    d = jnp.minimum(tf + 1.0 - p, 1.0)
    c0 = jnp.minimum(jnp.maximum(tf, 0.0), 127.0).astype(jnp.int32)
    c1 = jnp.minimum(jnp.maximum(tf + 1.0, 0.0), 127.0).astype(jnp.int32)
    return d, c0, c1, tf


def _gclamp(tf):
    c0 = jnp.minimum(jnp.maximum(tf, 0.0), 127.0).astype(jnp.int32)
    c1 = jnp.minimum(jnp.maximum(tf + 1.0, 0.0), 127.0).astype(jnp.int32)
    return c0, c1


def _body(I3, I2, F3, out3, win, flt, outt, fb_idx, fb_land,
          sem_in, sem_out, sem_fb):
    wid = lax.axis_index("s") * NC + lax.axis_index("c")
    iota = lax.iota(jnp.int32, 16)
    iota3 = iota * 3
    iota_f = iota.astype(jnp.float32)

    @pl.loop(0, TPW)
    def _tile(t):
        tile = wid * TPW + t
        b = tile // (NTI * NTJ)
        r = tile % (NTI * NTJ)
        i0 = (r // NTJ) * BI
        j0 = (r % NTJ) * BJ
        ylo = jnp.where(i0 == 0, 0, jnp.where(i0 == H - BI, YLO_MAX, i0 - HALO))
        xlo = jnp.where(j0 == 0, 0, jnp.where(j0 == W - BJ, YLO_MAX, j0 - HALO))

        cps = []
        for y in range(WY):
            src = I3.at[b, pl.ds(((ylo + y) * W + xlo) * D, WROW)]
            cps.append(pltpu.async_copy(src, win.at[pl.ds(y * WROW, WROW)], sem_in))
        for i in range(BI):
            src = F3.at[b, pl.ds(((i0 + i) * W + j0) * D * 3, FROW)]
            cps.append(pltpu.async_copy(src, flt.at[pl.ds(i * FROW, FROW)], sem_in))
        for c in cps:
            c.wait()

        ylo_f = ylo.astype(jnp.float32)
        xlo_f = xlo.astype(jnp.float32)
        y_lo_s = jnp.where(ylo > 0, 0.0, -jnp.inf)
        y_hi_s = jnp.where(ylo < YLO_MAX, float(WY - 1), jnp.inf)
        x_lo_s = jnp.where(xlo > 0, 0.0, -jnp.inf)
        x_hi_s = jnp.where(xlo < YLO_MAX, float(WX - 1), jnp.inf)
        bbase = b * (H * W * D)

        @pl.loop(0, NVEC)
        def _vec(v):
            i = v // (BJ * KV)
            rem = v % (BJ * KV)
            j = rem // KV
            k0 = (rem % KV) * 16
            fbase = ((i * BJ + j) * D + k0) * 3
            fy = plsc.load_gather(flt, [iota3 + fbase])
            fx = plsc.load_gather(flt, [iota3 + (fbase + 1)])
            fz = plsc.load_gather(flt, [iota3 + (fbase + 2)])
            yy = fy + (i0 + i).astype(jnp.float32)
            xx = fx + (j0 + j).astype(jnp.float32)
            zz = (fz + iota_f) + k0.astype(jnp.float32)

            dy, ly0, ly1, bady, tyf = _axis(yy, ylo_f, y_lo_s, y_hi_s, float(WY - 1))
            dx, lx0, lx1, badx, txf = _axis(xx, xlo_f, x_lo_s, x_hi_s, float(WX - 1))
            dz, z0, z1, _ = _zaxis(zz)

            sy0 = ly0 * WROW
            sy1 = ly1 * WROW
            sx0 = lx0 * D
            sx1 = lx1 * D
            c00 = sy0 + sx0
            c01 = sy0 + sx1
            c10 = sy1 + sx0
            c11 = sy1 + sx1
            ez = 1.0 - dz
            g = lambda idx: plsc.load_gather(win, [idx])
            v00 = g(c00 + z0) * dz + g(c00 + z1) * ez
            v01 = g(c01 + z0) * dz + g(c01 + z1) * ez
            v10 = g(c10 + z0) * dz + g(c10 + z1) * ez
            v11 = g(c11 + z0) * dz + g(c11 + z1) * ez
            ey = 1.0 - dy
            ex = 1.0 - dx
            res = ((v00 * dx + v01 * ex) * dy + (v10 * dx + v11 * ex) * ey)
            obase = (i * BJ + j) * D + k0
            outt[pl.ds(obase, 16)] = res

            @pl.when(jnp.any(bady | badx))
            def _fallback():
                gy0, gy1 = _gclamp(tyf)
                gx0, gx1 = _gclamp(txf)
                ry0 = gy0 * (W * D)
                ry1 = gy1 * (W * D)
                rx0 = gx0 * D
                rx1 = gx1 * D
                zb0 = z0 + bbase
                zb1 = z1 + bbase
                q = [ry0 + rx0 + zb0, ry0 + rx0 + zb1,
                     ry0 + rx1 + zb0, ry0 + rx1 + zb1,
                     ry1 + rx0 + zb0, ry1 + rx0 + zb1,
                     ry1 + rx1 + zb0, ry1 + rx1 + zb1]
                offs = []
                for c in range(8):
                    fb_idx[pl.ds(c * 16, 16)] = q[c] >> 7
                    offs.append(q[c] & 127)
                pltpu.async_copy(I2.at[fb_idx], fb_land, sem_fb).wait()
                vals = [plsc.load_gather(fb_land, [c * 16 + iota, offs[c]])
                        for c in range(8)]
                w00 = (vals[0] * dz + vals[1] * ez)
                w01 = (vals[2] * dz + vals[3] * ez)
                w10 = (vals[4] * dz + vals[5] * ez)
                w11 = (vals[6] * dz + vals[7] * ez)
                res2 = ((w00 * dx + w01 * ex) * dy + (w10 * dx + w11 * ex) * ey)
                outt[pl.ds(obase, 16)] = res2

        ocps = []
        for i in range(BI):
            dst = out3.at[b, pl.ds(((i0 + i) * W + j0) * D, OROW)]
            ocps.append(pltpu.async_copy(outt.at[pl.ds(i * OROW, OROW)], dst, sem_out))
        for c in ocps:
            c.wait()


@jax.jit
def _warp(I3, I2, F3):
    mesh = plsc.VectorSubcoreMesh(core_axis_name="c", subcore_axis_name="s",
                                  num_cores=NC, num_subcores=NS)
    return pl.kernel(
        _body,
        out_type=jax.ShapeDtypeStruct((B, H * W * D), jnp.float32),
        mesh=mesh,
        compiler_params=pltpu.CompilerParams(needs_layout_passes=False),
        scratch_types=[
            pltpu.VMEM((WSIZE,), jnp.float32),
            pltpu.VMEM((BI * FROW,), jnp.float32),
            pltpu.VMEM((BI * OROW,), jnp.float32),
            pltpu.VMEM((128,), jnp.int32),
            pltpu.VMEM((128, 128), jnp.float32),
            pltpu.SemaphoreType.DMA,
            pltpu.SemaphoreType.DMA,
            pltpu.SemaphoreType.DMA,
        ],
    )(I3, I2, F3)


def kernel(I, flow):
    I3 = I.reshape(B, H * W * D)
    I2 = I.reshape(B * H * W * D // 128, 128)
    F3 = flow.reshape(B, H * W * D * 3)
    out = _warp(I3, I2, F3)
    return out.reshape(B, H, W, D, 1)
